# baseline (speedup 1.0000x reference)
from functools import partial

import jax
import jax.numpy as jnp
from jax.experimental import pallas as pl
from jax.experimental.pallas import tpu as pltpu

_NC = 1024
_BE = 1280
_BN = 128


def _deg_kernel(dst_ref, dinv_ref, *, e_pad, n_pad):
    ne = e_pad // _BE
    nn = n_pad // _NC

    def n_body(j, _):
        n0 = j * _NC
        row = jax.lax.broadcasted_iota(jnp.int32, (_NC, 1), 0) + n0

        def e_body(i, acc):
            d = dst_ref[:, pl.ds(i * _BE, _BE)]
            m = (row == d).astype(jnp.float32)
            return acc + jnp.sum(m, axis=1, keepdims=True)

        acc = jax.lax.fori_loop(0, ne, e_body,
                                jnp.zeros((_NC, 1), jnp.float32))
        dinv_ref[pl.ds(n0, _NC), :] = jax.lax.rsqrt(acc + 1.0)
        return 0

    jax.lax.fori_loop(0, nn, n_body, 0)


def _mm_kernel(x_ref, w_ref, o_ref):
    o_ref[...] = jnp.dot(x_ref[...], w_ref[...],
                         preferred_element_type=jnp.float32)


def _agg_kernel(h_ref, dinv_ref, src_ref, dst_ref, b_ref, s_ref, t_ref,
                *rest, ne, n_pad, with_res):
    if with_res:
        rb_ref, res_ref, o_ref, hw_ref, acc_ref = rest
    else:
        o_ref, hw_ref, acc_ref = rest
    i = pl.program_id(0)
    nn = n_pad // _NC
    nh = h_ref.shape[1]

    @pl.when(i == 0)
    def _init():
        hw_ref[...] = h_ref[...] * dinv_ref[...]
        acc_ref[...] = jnp.zeros_like(acc_ref)

    s = src_ref[...]
    d = dst_ref[...]
    col = jax.lax.broadcasted_iota(jnp.int32, (1, _NC), 1)

    def g_body(j, g):
        n0 = j * _NC
        m = (s == col + n0).astype(jnp.float32)
        return g + jnp.dot(m, hw_ref[pl.ds(n0, _NC), :],
                           preferred_element_type=jnp.float32)

    g = jax.lax.fori_loop(0, nn, g_body,
                          jnp.zeros((_BE, nh), jnp.float32))
    rowi = jax.lax.broadcasted_iota(jnp.int32, (_NC, 1), 0)

    def s_body(j, _):
        n0 = j * _NC
        m = (rowi + n0 == d).astype(jnp.float32)
        acc_ref[pl.ds(n0, _NC), :] += jnp.dot(
            m, g, preferred_element_type=jnp.float32)
        return 0

    jax.lax.fori_loop(0, nn, s_body, 0)

    @pl.when(i == ne - 1)
    def _fin():
        agg = dinv_ref[...] * (acc_ref[...] + hw_ref[...]) + b_ref[...]
        y = jnp.maximum(agg * s_ref[...] + t_ref[...], 0.0)
        if with_res:
            y = y + res_ref[...] + rb_ref[...]
        o_ref[...] = y


def _cnn_kernel(x_ref, w1_ref, p_ref, w20_ref, w21_ref, w22_ref, xcT_ref,
                *, bn):
    t0 = w1_ref[:, 0:1]
    t1 = w1_ref[:, 1:2]
    t2 = w1_ref[:, 2:3]
    cb1 = p_ref[:, 0:1]
    s1 = p_ref[:, 1:2]
    be1 = p_ref[:, 2:3]
    cb2 = p_ref[:, 3:4]
    s2 = p_ref[:, 4:5]
    be2 = p_ref[:, 5:6]
    w20 = w20_ref[...]
    w21 = w21_ref[...]
    w22 = w22_ref[...]
    L = x_ref.shape[1]
    zc = jnp.zeros((1, 1), jnp.float32)
    z32 = jnp.zeros((w20.shape[0], 1), jnp.float32)

    lane = jax.lax.broadcasted_iota(jnp.int32, (1, bn), 1)

    def body(n, acc):
        xr = x_ref[pl.ds(n, 1), :]
        xm = jnp.concatenate([zc, xr[:, :L - 1]], axis=1)
        xp = jnp.concatenate([xr[:, 1:], zc], axis=1)
        y1 = t0 * xm + t1 * xr + t2 * xp + cb1
        y1 = jnp.maximum(y1 * s1 + be1, 0.0)
        y1m = jnp.concatenate([z32, y1[:, :L - 1]], axis=1)
        y1p = jnp.concatenate([y1[:, 1:], z32], axis=1)
        y2 = (jnp.dot(w20, y1m, preferred_element_type=jnp.float32)
              + jnp.dot(w21, y1, preferred_element_type=jnp.float32)
              + jnp.dot(w22, y1p, preferred_element_type=jnp.float32)
              + cb2)
        y2 = jnp.maximum(y2 * s2 + be2, 0.0)
        pooled = jnp.sum(y2, axis=1, keepdims=True) * (1.0 / L)
        oh = (lane == n).astype(jnp.float32)
        return acc + pooled * oh

    xcT_ref[...] = jax.lax.fori_loop(
        0, bn, body, jnp.zeros_like(xcT_ref))


def _cls_kernel(xg_ref, xc_ref, wg_ref, wc_ref, b_ref, o_ref):
    logits = (jnp.dot(xg_ref[...], wg_ref[...],
                      preferred_element_type=jnp.float32)
              + jnp.dot(xc_ref[...], wc_ref[...],
                        preferred_element_type=jnp.float32)
              + b_ref[...])
    m = jnp.max(logits, axis=1, keepdims=True)
    z = logits - m
    lse = jnp.log(jnp.sum(jnp.exp(z), axis=1, keepdims=True))
    o_ref[...] = z - lse


def _bn_scale(g):
    return g / jnp.sqrt(1.0 + 1e-5)


def kernel(x, edge_index, W1, b1, g1, be1, W2, b2, g2, be2, resW, resb,
           cw1, cb1, cg1, cbe1, cw2, cb2, cg2, cbe2, clsW, clsb):
    N, D = x.shape
    E = edge_index.shape[1]
    H = W1.shape[1]
    n_pad = -(-N // _NC) * _NC
    e_pad = -(-E // _BE) * _BE

    x_pad = jnp.pad(x, ((0, n_pad - N), (0, 0))) if n_pad > N else x
    src = edge_index[0]
    dst = edge_index[1]
    if e_pad > E:
        pad = jnp.full((e_pad - E,), -1, jnp.int32)
        src = jnp.concatenate([src, pad])
        dst = jnp.concatenate([dst, pad])
    src_col = src.reshape(e_pad, 1)
    dst_row = dst.reshape(1, e_pad)

    dinv = pl.pallas_call(
        partial(_deg_kernel, e_pad=e_pad, n_pad=n_pad),
        out_shape=jax.ShapeDtypeStruct((n_pad, 1), jnp.float32),
    )(dst_row)

    Wcat = jnp.concatenate([W1, resW], axis=1)
    xh = pl.pallas_call(
        _mm_kernel,
        out_shape=jax.ShapeDtypeStruct((n_pad, 2 * H), jnp.float32),
    )(x_pad, Wcat)
    h1 = xh[:, :H]
    res = xh[:, H:]

    ne = e_pad // _BE
    scratch = [pltpu.VMEM((n_pad, H), jnp.float32),
               pltpu.VMEM((n_pad, H), jnp.float32)]
    _full = lambda shape: pl.BlockSpec(shape, lambda i: (0, 0))
    base_specs = [
        _full((n_pad, H)),
        _full((n_pad, 1)),
        pl.BlockSpec((_BE, 1), lambda i: (i, 0)),
        pl.BlockSpec((1, _BE), lambda i: (0, i)),
        _full((1, H)), _full((1, H)), _full((1, H)),
    ]
    b1r = b1.reshape(1, H)
    s1r = _bn_scale(g1).reshape(1, H)
    t1r = be1.reshape(1, H)
    xg1 = pl.pallas_call(
        partial(_agg_kernel, ne=ne, n_pad=n_pad, with_res=False),
        grid=(ne,),
        in_specs=base_specs,
        out_specs=_full((n_pad, H)),
        out_shape=jax.ShapeDtypeStruct((n_pad, H), jnp.float32),
        scratch_shapes=scratch,
    )(h1, dinv, src_col, dst_row, b1r, s1r, t1r)

    h2 = pl.pallas_call(
        _mm_kernel,
        out_shape=jax.ShapeDtypeStruct((n_pad, H), jnp.float32),
    )(xg1, W2)

    b2r = b2.reshape(1, H)
    s2r = _bn_scale(g2).reshape(1, H)
    t2r = be2.reshape(1, H)
    rbr = resb.reshape(1, H)
    xg = pl.pallas_call(
        partial(_agg_kernel, ne=ne, n_pad=n_pad, with_res=True),
        grid=(ne,),
        in_specs=base_specs + [_full((1, H)), _full((n_pad, H))],
        out_specs=_full((n_pad, H)),
        out_shape=jax.ShapeDtypeStruct((n_pad, H), jnp.float32),
        scratch_shapes=scratch,
    )(h2, dinv, src_col, dst_row, b2r, s2r, t2r, rbr, res)

    C = cw1.shape[0]
    w1taps = cw1[:, 0, :]
    p = jnp.stack([cb1, _bn_scale(cg1), cbe1,
                   cb2, _bn_scale(cg2), cbe2], axis=1)
    w20 = cw2[:, :, 0]
    w21 = cw2[:, :, 1]
    w22 = cw2[:, :, 2]
    xcT = pl.pallas_call(
        partial(_cnn_kernel, bn=_BN),
        grid=(n_pad // _BN,),
        in_specs=[
            pl.BlockSpec((_BN, D), lambda i: (i, 0)),
            pl.BlockSpec((C, 3), lambda i: (0, 0)),
            pl.BlockSpec((C, 6), lambda i: (0, 0)),
            pl.BlockSpec((C, C), lambda i: (0, 0)),
            pl.BlockSpec((C, C), lambda i: (0, 0)),
            pl.BlockSpec((C, C), lambda i: (0, 0)),
        ],
        out_specs=pl.BlockSpec((C, _BN), lambda i: (0, i)),
        out_shape=jax.ShapeDtypeStruct((C, n_pad), jnp.float32),
    )(x_pad, w1taps, p, w20, w21, w22)
    xc = xcT.T

    out = pl.pallas_call(
        _cls_kernel,
        out_shape=jax.ShapeDtypeStruct((n_pad, clsW.shape[1]), jnp.float32),
    )(xg, xc, clsW[:H], clsW[H:], clsb.reshape(1, clsW.shape[1]))
    return out[:N]


# speedup vs baseline: 1.6135x; 1.6135x over previous
from functools import partial

import jax
import jax.numpy as jnp
from jax.experimental import pallas as pl
from jax.experimental.pallas import tpu as pltpu

_NC = 1024
_BE = 1280
_BN = 128


def _deg_kernel(dst_ref, dinv_ref, *, e_pad, n_pad):
    ne = e_pad // _BE
    nn = n_pad // _NC

    def n_body(j, _):
        n0 = j * _NC
        row = jax.lax.broadcasted_iota(jnp.int32, (_NC, 1), 0) + n0

        def e_body(i, acc):
            d = dst_ref[:, pl.ds(i * _BE, _BE)]
            m = (row == d).astype(jnp.float32)
            return acc + jnp.sum(m, axis=1, keepdims=True)

        acc = jax.lax.fori_loop(0, ne, e_body,
                                jnp.zeros((_NC, 1), jnp.float32))
        dinv_ref[pl.ds(n0, _NC), :] = jax.lax.rsqrt(acc + 1.0)
        return 0

    jax.lax.fori_loop(0, nn, n_body, 0)


def _mm_kernel(x_ref, w_ref, o_ref):
    o_ref[...] = jnp.dot(x_ref[...], w_ref[...],
                         preferred_element_type=jnp.float32)


def _agg_kernel(h_ref, dinv_ref, src_ref, dst_ref, b_ref, s_ref, t_ref,
                *rest, ne, n_pad, with_res):
    if with_res:
        rb_ref, res_ref, o_ref, hw_ref, acc_ref = rest
    else:
        o_ref, hw_ref, acc_ref = rest
    i = pl.program_id(0)
    nn = n_pad // _NC
    nh = h_ref.shape[1]

    @pl.when(i == 0)
    def _init():
        hw_ref[...] = h_ref[...] * dinv_ref[...]
        acc_ref[...] = jnp.zeros_like(acc_ref)

    s = src_ref[...]
    d = dst_ref[...]
    col = jax.lax.broadcasted_iota(jnp.int32, (1, _NC), 1)

    def g_body(j, g):
        n0 = j * _NC
        m = (s == col + n0).astype(jnp.float32)
        return g + jnp.dot(m, hw_ref[pl.ds(n0, _NC), :],
                           preferred_element_type=jnp.float32)

    g = jax.lax.fori_loop(0, nn, g_body,
                          jnp.zeros((_BE, nh), jnp.float32))
    rowi = jax.lax.broadcasted_iota(jnp.int32, (_NC, 1), 0)

    def s_body(j, _):
        n0 = j * _NC
        m = (rowi + n0 == d).astype(jnp.float32)
        acc_ref[pl.ds(n0, _NC), :] += jnp.dot(
            m, g, preferred_element_type=jnp.float32)
        return 0

    jax.lax.fori_loop(0, nn, s_body, 0)

    @pl.when(i == ne - 1)
    def _fin():
        agg = dinv_ref[...] * (acc_ref[...] + hw_ref[...]) + b_ref[...]
        y = jnp.maximum(agg * s_ref[...] + t_ref[...], 0.0)
        if with_res:
            y = y + res_ref[...] + rb_ref[...]
        o_ref[...] = y


def _cnn_kernel(x_ref, w1_ref, p_ref, w20_ref, w21_ref, w22_ref, xcT_ref,
                *, nb, L):
    t0 = w1_ref[:, 0:1]
    t1 = w1_ref[:, 1:2]
    t2 = w1_ref[:, 2:3]
    cb1 = p_ref[:, 0:1]
    s1 = p_ref[:, 1:2]
    be1 = p_ref[:, 2:3]
    cb2 = p_ref[:, 3:4]
    s2 = p_ref[:, 4:5]
    be2 = p_ref[:, 5:6]
    w20 = w20_ref[...]
    w21 = w21_ref[...]
    w22 = w22_ref[...]
    M = nb * L
    zc = jnp.zeros((1, 1), jnp.float32)
    z32 = jnp.zeros((w20.shape[0], 1), jnp.float32)
    col = jax.lax.broadcasted_iota(jnp.int32, (1, M), 1)
    mleft = (col % L != 0).astype(jnp.float32)
    mright = (col % L != L - 1).astype(jnp.float32)

    xr = x_ref[0]
    xm = jnp.concatenate([zc, xr[:, :M - 1]], axis=1) * mleft
    xp = jnp.concatenate([xr[:, 1:], zc], axis=1) * mright
    y1 = t0 * xm + t1 * xr + t2 * xp + cb1
    y1 = jnp.maximum(y1 * s1 + be1, 0.0)
    y1m = jnp.concatenate([z32, y1[:, :M - 1]], axis=1) * mleft
    y1p = jnp.concatenate([y1[:, 1:], z32], axis=1) * mright
    y2 = (jnp.dot(w20, y1m, preferred_element_type=jnp.float32)
          + jnp.dot(w21, y1, preferred_element_type=jnp.float32)
          + jnp.dot(w22, y1p, preferred_element_type=jnp.float32)
          + cb2)
    y2 = jnp.maximum(y2 * s2 + be2, 0.0)
    pooled = [jnp.sum(y2[:, k * L:(k + 1) * L], axis=1, keepdims=True)
              for k in range(nb)]
    xcT_ref[...] = jnp.concatenate(pooled, axis=1) * (1.0 / L)


def _cls_kernel(xg_ref, xc_ref, wg_ref, wc_ref, b_ref, o_ref):
    logits = (jnp.dot(xg_ref[...], wg_ref[...],
                      preferred_element_type=jnp.float32)
              + jnp.dot(xc_ref[...], wc_ref[...],
                        preferred_element_type=jnp.float32)
              + b_ref[...])
    m = jnp.max(logits, axis=1, keepdims=True)
    z = logits - m
    lse = jnp.log(jnp.sum(jnp.exp(z), axis=1, keepdims=True))
    o_ref[...] = z - lse


def _bn_scale(g):
    return g / jnp.sqrt(1.0 + 1e-5)


def kernel(x, edge_index, W1, b1, g1, be1, W2, b2, g2, be2, resW, resb,
           cw1, cb1, cg1, cbe1, cw2, cb2, cg2, cbe2, clsW, clsb):
    N, D = x.shape
    E = edge_index.shape[1]
    H = W1.shape[1]
    n_pad = -(-N // _NC) * _NC
    e_pad = -(-E // _BE) * _BE

    x_pad = jnp.pad(x, ((0, n_pad - N), (0, 0))) if n_pad > N else x
    src = edge_index[0]
    dst = edge_index[1]
    if e_pad > E:
        pad = jnp.full((e_pad - E,), -1, jnp.int32)
        src = jnp.concatenate([src, pad])
        dst = jnp.concatenate([dst, pad])
    src_col = src.reshape(e_pad, 1)
    dst_row = dst.reshape(1, e_pad)

    dinv = pl.pallas_call(
        partial(_deg_kernel, e_pad=e_pad, n_pad=n_pad),
        out_shape=jax.ShapeDtypeStruct((n_pad, 1), jnp.float32),
    )(dst_row)

    Wcat = jnp.concatenate([W1, resW], axis=1)
    xh = pl.pallas_call(
        _mm_kernel,
        out_shape=jax.ShapeDtypeStruct((n_pad, 2 * H), jnp.float32),
    )(x_pad, Wcat)
    h1 = xh[:, :H]
    res = xh[:, H:]

    ne = e_pad // _BE
    scratch = [pltpu.VMEM((n_pad, H), jnp.float32),
               pltpu.VMEM((n_pad, H), jnp.float32)]
    _full = lambda shape: pl.BlockSpec(shape, lambda i: (0, 0))
    base_specs = [
        _full((n_pad, H)),
        _full((n_pad, 1)),
        pl.BlockSpec((_BE, 1), lambda i: (i, 0)),
        pl.BlockSpec((1, _BE), lambda i: (0, i)),
        _full((1, H)), _full((1, H)), _full((1, H)),
    ]
    b1r = b1.reshape(1, H)
    s1r = _bn_scale(g1).reshape(1, H)
    t1r = be1.reshape(1, H)
    xg1 = pl.pallas_call(
        partial(_agg_kernel, ne=ne, n_pad=n_pad, with_res=False),
        grid=(ne,),
        in_specs=base_specs,
        out_specs=_full((n_pad, H)),
        out_shape=jax.ShapeDtypeStruct((n_pad, H), jnp.float32),
        scratch_shapes=scratch,
    )(h1, dinv, src_col, dst_row, b1r, s1r, t1r)

    h2 = pl.pallas_call(
        _mm_kernel,
        out_shape=jax.ShapeDtypeStruct((n_pad, H), jnp.float32),
    )(xg1, W2)

    b2r = b2.reshape(1, H)
    s2r = _bn_scale(g2).reshape(1, H)
    t2r = be2.reshape(1, H)
    rbr = resb.reshape(1, H)
    xg = pl.pallas_call(
        partial(_agg_kernel, ne=ne, n_pad=n_pad, with_res=True),
        grid=(ne,),
        in_specs=base_specs + [_full((1, H)), _full((n_pad, H))],
        out_specs=_full((n_pad, H)),
        out_shape=jax.ShapeDtypeStruct((n_pad, H), jnp.float32),
        scratch_shapes=scratch,
    )(h2, dinv, src_col, dst_row, b2r, s2r, t2r, rbr, res)

    C = cw1.shape[0]
    w1taps = cw1[:, 0, :]
    p = jnp.stack([cb1, _bn_scale(cg1), cbe1,
                   cb2, _bn_scale(cg2), cbe2], axis=1)
    w20 = cw2[:, :, 0]
    w21 = cw2[:, :, 1]
    w22 = cw2[:, :, 2]
    x_flat = x_pad.reshape(n_pad // _BN, 1, _BN * D)
    xcT = pl.pallas_call(
        partial(_cnn_kernel, nb=_BN, L=D),
        grid=(n_pad // _BN,),
        in_specs=[
            pl.BlockSpec((1, 1, _BN * D), lambda i: (i, 0, 0)),
            pl.BlockSpec((C, 3), lambda i: (0, 0)),
            pl.BlockSpec((C, 6), lambda i: (0, 0)),
            pl.BlockSpec((C, C), lambda i: (0, 0)),
            pl.BlockSpec((C, C), lambda i: (0, 0)),
            pl.BlockSpec((C, C), lambda i: (0, 0)),
        ],
        out_specs=pl.BlockSpec((C, _BN), lambda i: (0, i)),
        out_shape=jax.ShapeDtypeStruct((C, n_pad), jnp.float32),
    )(x_flat, w1taps, p, w20, w21, w22)
    xc = xcT.T

    out = pl.pallas_call(
        _cls_kernel,
        out_shape=jax.ShapeDtypeStruct((n_pad, clsW.shape[1]), jnp.float32),
    )(xg, xc, clsW[:H], clsW[H:], clsb.reshape(1, clsW.shape[1]))
    return out[:N]
